# baseline (speedup 1.0000x reference)
import jax
from jax.experimental import pallas as pl
from jax.experimental.pallas import tpu as pltpu

_BLOCKS = (1024, 1024, 2048)


def kernel(tokens, W_pos):
    batch, seq_len = tokens.shape
    d_model = W_pos.shape[1]
    assert sum(_BLOCKS) == seq_len
    offs = [0]
    for s in _BLOCKS:
        offs.append(offs[-1] + s)
    nblk = len(_BLOCKS)

    def _dma_kernel(w_hbm, o_hbm, buf, in_sems, out_sems):
        def in_copy(i):
            return pltpu.make_async_copy(
                w_hbm.at[pl.ds(offs[i], _BLOCKS[i])],
                buf.at[pl.ds(offs[i], _BLOCKS[i])],
                in_sems.at[i],
            )

        def out_copy(i, b):
            return pltpu.make_async_copy(
                buf.at[pl.ds(offs[i], _BLOCKS[i])],
                o_hbm.at[b, pl.ds(offs[i], _BLOCKS[i])],
                out_sems.at[i, b],
            )

        for i in range(nblk):
            in_copy(i).start()
        for i in range(nblk):
            in_copy(i).wait()
            for b in range(batch):
                out_copy(i, b).start()
        for i in range(nblk):
            for b in range(batch):
                out_copy(i, b).wait()

    out = pl.pallas_call(
        _dma_kernel,
        in_specs=[pl.BlockSpec(memory_space=pl.ANY)],
        out_specs=pl.BlockSpec(memory_space=pl.ANY),
        out_shape=jax.ShapeDtypeStruct((batch, seq_len, d_model), W_pos.dtype),
        scratch_shapes=[
            pltpu.VMEM((seq_len, d_model), W_pos.dtype),
            pltpu.SemaphoreType.DMA((nblk,)),
            pltpu.SemaphoreType.DMA((nblk, batch)),
        ],
    )(W_pos)
    return out
